# Initial kernel scaffold; baseline (speedup 1.0000x reference)
import jax
import jax.numpy as jnp
from jax.experimental import pallas as pl


def kernel(x, edge_index, edge_type, batch, w1_rel, w1_root, b1, w2_rel, w2_root, b2, lin_w, lin_b):
    raise NotImplementedError("write your pallas kernel here")


# speedup vs baseline: 2.4011x; 2.4011x over previous
import dataclasses
import functools

import jax
import jax.numpy as jnp
from jax import lax
from jax.experimental import pallas as pl
from jax.experimental.pallas import tpu as pltpu
from jax.experimental.pallas import tpu_sc as plsc

N_NODES = 10000
N_EDGES = 320000
DIM = 128
NREL = 3
NGRAPH = 128
CLASSES = 64

NC = 2
NS = 16
PER_W = N_EDGES // (NC * NS)
CH = 80
NCHUNK = PER_W // CH
ROWS_PER_S = (N_NODES // NS) // 8 * 8
ROWS_REM = N_NODES - NS * ROWS_PER_S

_mesh = plsc.VectorSubcoreMesh(core_axis_name="c", subcore_axis_name="s")

_sc_params = pltpu.CompilerParams()
if "needs_layout_passes" in pltpu.CompilerParams.__dataclass_fields__:
    _sc_params = dataclasses.replace(_sc_params, needs_layout_passes=False)


@functools.partial(
    pl.kernel,
    out_type=jax.ShapeDtypeStruct((NC, N_NODES, DIM), jnp.float32),
    mesh=_mesh,
    scratch_types=[
        pltpu.VMEM_SHARED((N_NODES, DIM), jnp.float32),
        pltpu.VMEM((CH,), jnp.int32),
        pltpu.VMEM((CH,), jnp.int32),
        pltpu.VMEM((CH, DIM), jnp.float32),
    ],
    compiler_params=_sc_params,
)
def _sc_counts(didx_hbm, et_hbm, zeros_hbm, out_hbm, acc_sh, di_v, et_v,
               rows_v):
    c = lax.axis_index("c")
    s = lax.axis_index("s")
    base = (c * NS + s) * PER_W
    lanes = lax.iota(jnp.int32, 16)
    ones16 = jnp.ones((16,), jnp.float32)
    zeros16 = jnp.zeros((16,), jnp.float32)

    pltpu.sync_copy(zeros_hbm.at[pl.ds(0, CH)], rows_v)

    @pl.when(s == 0)
    def _():
        pltpu.sync_copy(zeros_hbm, acc_sh)

    plsc.subcore_barrier()

    @pl.loop(0, NCHUNK)
    def _(k):
        off = base + k * CH
        pltpu.sync_copy(didx_hbm.at[pl.ds(off, CH)], di_v)
        pltpu.sync_copy(et_hbm.at[pl.ds(off, CH)], et_v)

        @pl.loop(0, CH)
        def _(i):
            ri = jnp.full((16,), i, jnp.int32)
            t16 = plsc.load_gather(et_v, [ri])
            plsc.store_scatter(rows_v, [ri, t16 * 16 + lanes], ones16)

        pltpu.sync_copy(rows_v, acc_sh.at[di_v], add=True)

        @pl.loop(0, CH)
        def _(i):
            ri = jnp.full((16,), i, jnp.int32)
            t16 = plsc.load_gather(et_v, [ri])
            plsc.store_scatter(rows_v, [ri, t16 * 16 + lanes], zeros16)

    plsc.subcore_barrier()
    r0 = s * ROWS_PER_S
    pltpu.sync_copy(acc_sh.at[pl.ds(r0, ROWS_PER_S)],
                    out_hbm.at[c, pl.ds(r0, ROWS_PER_S)])

    @pl.when(s == 0)
    def _():
        rr = NS * ROWS_PER_S
        pltpu.sync_copy(acc_sh.at[pl.ds(rr, ROWS_REM)],
                        out_hbm.at[c, pl.ds(rr, ROWS_REM)])


@functools.partial(
    pl.kernel,
    out_type=jax.ShapeDtypeStruct((NC, N_NODES, DIM), jnp.float32),
    mesh=_mesh,
    scratch_types=[
        pltpu.VMEM_SHARED((N_NODES, DIM), jnp.float32),
        pltpu.VMEM((CH,), jnp.int32),
        pltpu.VMEM((CH,), jnp.int32),
        pltpu.VMEM((CH,), jnp.int32),
        pltpu.VMEM((CH, DIM), jnp.float32),
        pltpu.VMEM((NREL * N_NODES,), jnp.float32),
    ],
    compiler_params=_sc_params,
)
def _sc_agg(hs_hbm, gidx_hbm, cidx_hbm, didx_hbm, stbl_hbm, zeros_hbm,
            out_hbm, acc_sh, gi_v, ci_v, di_v, rows_v, stbl_v):
    c = lax.axis_index("c")
    s = lax.axis_index("s")
    base = (c * NS + s) * PER_W
    lanes = lax.iota(jnp.int32, 16)

    pltpu.sync_copy(stbl_hbm, stbl_v)

    @pl.when(s == 0)
    def _():
        pltpu.sync_copy(zeros_hbm, acc_sh)

    plsc.subcore_barrier()

    @pl.loop(0, NCHUNK)
    def _(k):
        off = base + k * CH
        pltpu.sync_copy(gidx_hbm.at[pl.ds(off, CH)], gi_v)
        pltpu.sync_copy(cidx_hbm.at[pl.ds(off, CH)], ci_v)
        pltpu.sync_copy(didx_hbm.at[pl.ds(off, CH)], di_v)
        pltpu.sync_copy(hs_hbm.at[gi_v], rows_v)

        for g in range(CH // 16):
            e16 = g * 16 + lanes
            ci16 = ci_v[pl.ds(g * 16, 16)]
            sv16 = plsc.load_gather(stbl_v, [ci16])

            @pl.loop(0, DIM // 4)
            def _(j):
                for jj in range(4):
                    cj = jnp.full((16,), j * 4 + jj, jnp.int32)
                    v = plsc.load_gather(rows_v, [e16, cj])
                    plsc.store_scatter(rows_v, [e16, cj], v * sv16)

        pltpu.sync_copy(rows_v, acc_sh.at[di_v], add=True)

    plsc.subcore_barrier()
    r0 = s * ROWS_PER_S
    pltpu.sync_copy(acc_sh.at[pl.ds(r0, ROWS_PER_S)],
                    out_hbm.at[c, pl.ds(r0, ROWS_PER_S)])

    @pl.when(s == 0)
    def _():
        rr = NS * ROWS_PER_S
        pltpu.sync_copy(acc_sh.at[pl.ds(rr, ROWS_REM)],
                        out_hbm.at[c, pl.ds(rr, ROWS_REM)])


def _mm_rel_body(x_ref, w_ref, o_ref):
    o_ref[...] = jnp.dot(x_ref[...], w_ref[0], preferred_element_type=jnp.float32)


_mm_rel = pl.pallas_call(
    _mm_rel_body,
    grid=(NREL,),
    in_specs=[pl.BlockSpec((N_NODES, DIM), lambda r: (0, 0)),
              pl.BlockSpec((1, DIM, DIM), lambda r: (r, 0, 0))],
    out_specs=pl.BlockSpec((N_NODES, DIM), lambda r: (r, 0)),
    out_shape=jax.ShapeDtypeStruct((NREL * N_NODES, DIM), jnp.float32),
)


def _recip_body(c_ref, o_ref):
    o_ref[...] = 1.0 / jnp.maximum(c_ref[0] + c_ref[1], 1.0)


_recip = pl.pallas_call(
    _recip_body,
    in_specs=[pl.BlockSpec((NC, N_NODES, DIM), lambda: (0, 0, 0))],
    out_specs=pl.BlockSpec((N_NODES, DIM), lambda: (0, 0)),
    out_shape=jax.ShapeDtypeStruct((N_NODES, DIM), jnp.float32),
)


def _mid_body(x_ref, wr_ref, b_ref, a0_ref, a1_ref, wrel_ref, h_ref, hs_ref):
    r = pl.program_id(0)

    @pl.when(r == 0)
    def _():
        h_ref[...] = jnp.maximum(
            jnp.dot(x_ref[...], wr_ref[...], preferred_element_type=jnp.float32)
            + b_ref[...] + a0_ref[...] + a1_ref[...], 0.0)

    hs_ref[...] = jnp.dot(h_ref[...], wrel_ref[0],
                          preferred_element_type=jnp.float32)


_mid = pl.pallas_call(
    _mid_body,
    grid=(NREL,),
    in_specs=[
        pl.BlockSpec((N_NODES, DIM), lambda r: (0, 0)),
        pl.BlockSpec((DIM, DIM), lambda r: (0, 0)),
        pl.BlockSpec((1, DIM), lambda r: (0, 0)),
        pl.BlockSpec((N_NODES, DIM), lambda r: (0, 0)),
        pl.BlockSpec((N_NODES, DIM), lambda r: (0, 0)),
        pl.BlockSpec((1, DIM, DIM), lambda r: (r, 0, 0)),
    ],
    out_specs=[pl.BlockSpec((N_NODES, DIM), lambda r: (0, 0)),
               pl.BlockSpec((N_NODES, DIM), lambda r: (r, 0))],
    out_shape=[jax.ShapeDtypeStruct((N_NODES, DIM), jnp.float32),
               jax.ShapeDtypeStruct((NREL * N_NODES, DIM), jnp.float32)],
)


def _final_body(h1_ref, wr_ref, b_ref, a0_ref, a1_ref, g_ref, lw_ref, lb_ref,
                o_ref):
    h2 = jnp.maximum(
        jnp.dot(h1_ref[...], wr_ref[...], preferred_element_type=jnp.float32)
        + b_ref[...] + a0_ref[...] + a1_ref[...], 0.0)
    gids = lax.broadcasted_iota(jnp.int32, (1, NGRAPH), 1)
    p = (g_ref[...] == gids).astype(jnp.float32)
    sums = lax.dot_general(p, h2, (((0,), (0,)), ((), ())),
                           preferred_element_type=jnp.float32)
    cnts = jnp.sum(p, axis=0)
    pooled = sums / jnp.maximum(cnts, 1.0)[:, None]
    o_ref[...] = (jnp.dot(pooled, lw_ref[...], preferred_element_type=jnp.float32)
                  + lb_ref[...])


_final = pl.pallas_call(
    _final_body,
    in_specs=[
        pl.BlockSpec((N_NODES, DIM), lambda: (0, 0)),
        pl.BlockSpec((DIM, DIM), lambda: (0, 0)),
        pl.BlockSpec((1, DIM), lambda: (0, 0)),
        pl.BlockSpec((N_NODES, DIM), lambda: (0, 0)),
        pl.BlockSpec((N_NODES, DIM), lambda: (0, 0)),
        pl.BlockSpec((N_NODES, 1), lambda: (0, 0)),
        pl.BlockSpec((DIM, CLASSES), lambda: (0, 0)),
        pl.BlockSpec((1, CLASSES), lambda: (0, 0)),
    ],
    out_specs=pl.BlockSpec((NGRAPH, CLASSES), lambda: (0, 0)),
    out_shape=jax.ShapeDtypeStruct((NGRAPH, CLASSES), jnp.float32),
)


def kernel(x, edge_index, edge_type, batch, w1_rel, w1_root, b1,
           w2_rel, w2_root, b2, lin_w, lin_b):
    src = edge_index[0].astype(jnp.int32)
    dst = edge_index[1].astype(jnp.int32)
    et = edge_type.astype(jnp.int32)
    gidx = et * N_NODES + src
    cidx = et * N_NODES + dst
    zeros_agg = jnp.zeros((N_NODES, DIM), jnp.float32)

    cnt_parts = _sc_counts(dst, et, zeros_agg)
    s2d = _recip(cnt_parts)
    stbl = jnp.concatenate([s2d[:, 16 * r] for r in range(NREL)])
    hs1 = _mm_rel(x, w1_rel)
    agg1 = _sc_agg(hs1, gidx, cidx, dst, stbl, zeros_agg)
    h1, hs2 = _mid(x, w1_root, b1.reshape(1, DIM), agg1[0], agg1[1], w2_rel)
    agg2 = _sc_agg(hs2, gidx, cidx, dst, stbl, zeros_agg)
    return _final(h1, w2_root, b2.reshape(1, DIM), agg2[0], agg2[1],
                  batch.astype(jnp.int32).reshape(N_NODES, 1),
                  lin_w, lin_b.reshape(1, CLASSES))


# speedup vs baseline: 2.5660x; 1.0687x over previous
import dataclasses
import functools

import jax
import jax.numpy as jnp
from jax import lax
from jax.experimental import pallas as pl
from jax.experimental.pallas import tpu as pltpu
from jax.experimental.pallas import tpu_sc as plsc

N_NODES = 10000
N_EDGES = 320000
DIM = 128
NREL = 3
NGRAPH = 128
CLASSES = 64

NC = 2
NS = 16
PER_W = N_EDGES // (NC * NS)
CH = 80
NCHUNK = PER_W // CH
ROWS_PER_S = (N_NODES // NS) // 8 * 8
ROWS_REM = N_NODES - NS * ROWS_PER_S

_mesh = plsc.VectorSubcoreMesh(core_axis_name="c", subcore_axis_name="s")

_sc_params = pltpu.CompilerParams()
if "needs_layout_passes" in pltpu.CompilerParams.__dataclass_fields__:
    _sc_params = dataclasses.replace(_sc_params, needs_layout_passes=False)


@functools.partial(
    pl.kernel,
    out_type=jax.ShapeDtypeStruct((NC, N_NODES, DIM), jnp.float32),
    mesh=_mesh,
    scratch_types=[
        pltpu.VMEM_SHARED((N_NODES, DIM), jnp.float32),
        pltpu.VMEM((CH,), jnp.int32),
        pltpu.VMEM((CH,), jnp.int32),
        pltpu.VMEM((CH, DIM), jnp.float32),
    ],
    compiler_params=_sc_params,
)
def _sc_counts(didx_hbm, et_hbm, zeros_hbm, out_hbm, acc_sh, di_v, et_v,
               rows_v):
    c = lax.axis_index("c")
    s = lax.axis_index("s")
    base = (c * NS + s) * PER_W
    lanes = lax.iota(jnp.int32, 16)
    ones16 = jnp.ones((16,), jnp.float32)
    zeros16 = jnp.zeros((16,), jnp.float32)

    pltpu.sync_copy(zeros_hbm.at[pl.ds(0, CH)], rows_v)

    @pl.when(s == 0)
    def _():
        pltpu.sync_copy(zeros_hbm, acc_sh)

    plsc.subcore_barrier()

    @pl.loop(0, NCHUNK)
    def _(k):
        off = base + k * CH
        pltpu.sync_copy(didx_hbm.at[pl.ds(off, CH)], di_v)
        pltpu.sync_copy(et_hbm.at[pl.ds(off, CH)], et_v)

        @pl.loop(0, CH)
        def _(i):
            ri = jnp.full((16,), i, jnp.int32)
            t16 = plsc.load_gather(et_v, [ri])
            plsc.store_scatter(rows_v, [ri, t16 * 16 + lanes], ones16)

        pltpu.sync_copy(rows_v, acc_sh.at[di_v], add=True)

        @pl.loop(0, CH)
        def _(i):
            ri = jnp.full((16,), i, jnp.int32)
            t16 = plsc.load_gather(et_v, [ri])
            plsc.store_scatter(rows_v, [ri, t16 * 16 + lanes], zeros16)

    plsc.subcore_barrier()
    r0 = s * ROWS_PER_S
    pltpu.sync_copy(acc_sh.at[pl.ds(r0, ROWS_PER_S)],
                    out_hbm.at[c, pl.ds(r0, ROWS_PER_S)])

    @pl.when(s == 0)
    def _():
        rr = NS * ROWS_PER_S
        pltpu.sync_copy(acc_sh.at[pl.ds(rr, ROWS_REM)],
                        out_hbm.at[c, pl.ds(rr, ROWS_REM)])


@functools.partial(
    pl.kernel,
    out_type=jax.ShapeDtypeStruct((NC, N_NODES, DIM), jnp.float32),
    mesh=_mesh,
    scratch_types=[
        pltpu.VMEM_SHARED((N_NODES, DIM), jnp.float32),
        pltpu.VMEM((CH,), jnp.int32),
        pltpu.VMEM((CH,), jnp.int32),
        pltpu.VMEM((CH,), jnp.int32),
        pltpu.VMEM((CH,), jnp.int32),
        pltpu.VMEM((CH,), jnp.int32),
        pltpu.VMEM((CH,), jnp.int32),
        pltpu.VMEM((CH,), jnp.float32),
        pltpu.VMEM((CH,), jnp.float32),
        pltpu.VMEM((CH, DIM), jnp.float32),
        pltpu.VMEM((CH, DIM), jnp.float32),
        pltpu.SemaphoreType.DMA,
        pltpu.SemaphoreType.DMA,
        pltpu.SemaphoreType.DMA,
        pltpu.SemaphoreType.DMA,
    ],
    compiler_params=_sc_params,
)
def _sc_agg(hs_hbm, gidx_hbm, cidx_hbm, didx_hbm, stbl_hbm, zeros_hbm,
            out_hbm, acc_sh, gi0_v, ci0_v, di0_v, gi1_v, ci1_v, di1_v,
            sv0_v, sv1_v, rows0_v, rows1_v, semr0, semr1, sems0, sems1):
    c = lax.axis_index("c")
    s = lax.axis_index("s")
    base = (c * NS + s) * PER_W
    lanes = lax.iota(jnp.int32, 16)

    @pl.when(s == 0)
    def _():
        pltpu.sync_copy(zeros_hbm, acc_sh)

    plsc.subcore_barrier()

    def fetch(k, gi_v, ci_v, di_v, sv_v, rows_v, semr, sems):
        off = base + k * CH
        pltpu.sync_copy(gidx_hbm.at[pl.ds(off, CH)], gi_v)
        pltpu.async_copy(hs_hbm.at[gi_v], rows_v, semr)
        pltpu.sync_copy(cidx_hbm.at[pl.ds(off, CH)], ci_v)
        pltpu.async_copy(stbl_hbm.at[ci_v], sv_v, sems)
        pltpu.sync_copy(didx_hbm.at[pl.ds(off, CH)], di_v)

    def process(gi_v, ci_v, di_v, sv_v, rows_v, semr, sems):
        pltpu.make_async_copy(hs_hbm.at[gi_v], rows_v, semr).wait()
        pltpu.make_async_copy(stbl_hbm.at[ci_v], sv_v, sems).wait()
        for g in range(CH // 16):
            e16 = g * 16 + lanes
            sv16 = sv_v[pl.ds(g * 16, 16)]

            @pl.loop(0, DIM // 8)
            def _(j):
                c0 = jnp.full((16,), j * 8, jnp.int32)
                for jj in range(8):
                    cj = c0 + jj
                    v = plsc.load_gather(rows_v, [e16, cj])
                    plsc.store_scatter(rows_v, [e16, cj], v * sv16)

        pltpu.sync_copy(rows_v, acc_sh.at[di_v], add=True)

    fetch(0, gi0_v, ci0_v, di0_v, sv0_v, rows0_v, semr0, sems0)

    @pl.loop(0, (NCHUNK - 1) // 2)
    def _(j):
        fetch(2 * j + 1, gi1_v, ci1_v, di1_v, sv1_v, rows1_v, semr1, sems1)
        process(gi0_v, ci0_v, di0_v, sv0_v, rows0_v, semr0, sems0)
        fetch(2 * j + 2, gi0_v, ci0_v, di0_v, sv0_v, rows0_v, semr0, sems0)
        process(gi1_v, ci1_v, di1_v, sv1_v, rows1_v, semr1, sems1)

    process(gi0_v, ci0_v, di0_v, sv0_v, rows0_v, semr0, sems0)

    plsc.subcore_barrier()
    r0 = s * ROWS_PER_S
    pltpu.sync_copy(acc_sh.at[pl.ds(r0, ROWS_PER_S)],
                    out_hbm.at[c, pl.ds(r0, ROWS_PER_S)])

    @pl.when(s == 0)
    def _():
        rr = NS * ROWS_PER_S
        pltpu.sync_copy(acc_sh.at[pl.ds(rr, ROWS_REM)],
                        out_hbm.at[c, pl.ds(rr, ROWS_REM)])


def _mm_rel_body(x_ref, w_ref, o_ref):
    o_ref[...] = jnp.dot(x_ref[...], w_ref[0], preferred_element_type=jnp.float32)


_mm_rel = pl.pallas_call(
    _mm_rel_body,
    grid=(NREL,),
    in_specs=[pl.BlockSpec((N_NODES, DIM), lambda r: (0, 0)),
              pl.BlockSpec((1, DIM, DIM), lambda r: (r, 0, 0))],
    out_specs=pl.BlockSpec((N_NODES, DIM), lambda r: (r, 0)),
    out_shape=jax.ShapeDtypeStruct((NREL * N_NODES, DIM), jnp.float32),
)


def _recip_body(c_ref, o_ref):
    o_ref[...] = 1.0 / jnp.maximum(c_ref[0] + c_ref[1], 1.0)


_recip = pl.pallas_call(
    _recip_body,
    in_specs=[pl.BlockSpec((NC, N_NODES, DIM), lambda: (0, 0, 0))],
    out_specs=pl.BlockSpec((N_NODES, DIM), lambda: (0, 0)),
    out_shape=jax.ShapeDtypeStruct((N_NODES, DIM), jnp.float32),
)


def _mid_body(x_ref, wr_ref, b_ref, a0_ref, a1_ref, wrel_ref, h_ref, hs_ref):
    r = pl.program_id(0)

    @pl.when(r == 0)
    def _():
        h_ref[...] = jnp.maximum(
            jnp.dot(x_ref[...], wr_ref[...], preferred_element_type=jnp.float32)
            + b_ref[...] + a0_ref[...] + a1_ref[...], 0.0)

    hs_ref[...] = jnp.dot(h_ref[...], wrel_ref[0],
                          preferred_element_type=jnp.float32)


_mid = pl.pallas_call(
    _mid_body,
    grid=(NREL,),
    in_specs=[
        pl.BlockSpec((N_NODES, DIM), lambda r: (0, 0)),
        pl.BlockSpec((DIM, DIM), lambda r: (0, 0)),
        pl.BlockSpec((1, DIM), lambda r: (0, 0)),
        pl.BlockSpec((N_NODES, DIM), lambda r: (0, 0)),
        pl.BlockSpec((N_NODES, DIM), lambda r: (0, 0)),
        pl.BlockSpec((1, DIM, DIM), lambda r: (r, 0, 0)),
    ],
    out_specs=[pl.BlockSpec((N_NODES, DIM), lambda r: (0, 0)),
               pl.BlockSpec((N_NODES, DIM), lambda r: (r, 0))],
    out_shape=[jax.ShapeDtypeStruct((N_NODES, DIM), jnp.float32),
               jax.ShapeDtypeStruct((NREL * N_NODES, DIM), jnp.float32)],
)


def _final_body(h1_ref, wr_ref, b_ref, a0_ref, a1_ref, g_ref, lw_ref, lb_ref,
                o_ref):
    h2 = jnp.maximum(
        jnp.dot(h1_ref[...], wr_ref[...], preferred_element_type=jnp.float32)
        + b_ref[...] + a0_ref[...] + a1_ref[...], 0.0)
    gids = lax.broadcasted_iota(jnp.int32, (1, NGRAPH), 1)
    p = (g_ref[...] == gids).astype(jnp.float32)
    sums = lax.dot_general(p, h2, (((0,), (0,)), ((), ())),
                           preferred_element_type=jnp.float32)
    cnts = jnp.sum(p, axis=0)
    pooled = sums / jnp.maximum(cnts, 1.0)[:, None]
    o_ref[...] = (jnp.dot(pooled, lw_ref[...], preferred_element_type=jnp.float32)
                  + lb_ref[...])


_final = pl.pallas_call(
    _final_body,
    in_specs=[
        pl.BlockSpec((N_NODES, DIM), lambda: (0, 0)),
        pl.BlockSpec((DIM, DIM), lambda: (0, 0)),
        pl.BlockSpec((1, DIM), lambda: (0, 0)),
        pl.BlockSpec((N_NODES, DIM), lambda: (0, 0)),
        pl.BlockSpec((N_NODES, DIM), lambda: (0, 0)),
        pl.BlockSpec((N_NODES, 1), lambda: (0, 0)),
        pl.BlockSpec((DIM, CLASSES), lambda: (0, 0)),
        pl.BlockSpec((1, CLASSES), lambda: (0, 0)),
    ],
    out_specs=pl.BlockSpec((NGRAPH, CLASSES), lambda: (0, 0)),
    out_shape=jax.ShapeDtypeStruct((NGRAPH, CLASSES), jnp.float32),
)


def kernel(x, edge_index, edge_type, batch, w1_rel, w1_root, b1,
           w2_rel, w2_root, b2, lin_w, lin_b):
    src = edge_index[0].astype(jnp.int32)
    dst = edge_index[1].astype(jnp.int32)
    et = edge_type.astype(jnp.int32)
    gidx = et * N_NODES + src
    cidx = et * N_NODES + dst
    zeros_agg = jnp.zeros((N_NODES, DIM), jnp.float32)

    cnt_parts = _sc_counts(dst, et, zeros_agg)
    s2d = _recip(cnt_parts)
    stbl = jnp.concatenate([s2d[:, 16 * r] for r in range(NREL)])
    hs1 = _mm_rel(x, w1_rel)
    agg1 = _sc_agg(hs1, gidx, cidx, dst, stbl, zeros_agg)
    h1, hs2 = _mid(x, w1_root, b1.reshape(1, DIM), agg1[0], agg1[1], w2_rel)
    agg2 = _sc_agg(hs2, gidx, cidx, dst, stbl, zeros_agg)
    return _final(h1, w2_root, b2.reshape(1, DIM), agg2[0], agg2[1],
                  batch.astype(jnp.int32).reshape(N_NODES, 1),
                  lin_w, lin_b.reshape(1, CLASSES))


# speedup vs baseline: 5.9245x; 2.3088x over previous
import dataclasses
import functools

import jax
import jax.numpy as jnp
from jax import lax
from jax.experimental import pallas as pl
from jax.experimental.pallas import tpu as pltpu
from jax.experimental.pallas import tpu_sc as plsc

N_NODES = 10000
N_EDGES = 320000
DIM = 128
NREL = 3
NGRAPH = 128
CLASSES = 64

NC = 2
NS = 16
PER_W = N_EDGES // (NC * NS)
CH = 80
NCHUNK = PER_W // CH
ROWS_PER_S = (N_NODES // NS) // 8 * 8
ROWS_REM = N_NODES - NS * ROWS_PER_S

_mesh = plsc.VectorSubcoreMesh(core_axis_name="c", subcore_axis_name="s")

_sc_params = pltpu.CompilerParams()
if "needs_layout_passes" in pltpu.CompilerParams.__dataclass_fields__:
    _sc_params = dataclasses.replace(_sc_params, needs_layout_passes=False)


@functools.partial(
    pl.kernel,
    out_type=jax.ShapeDtypeStruct((NC, N_NODES, DIM), jnp.float32),
    mesh=_mesh,
    scratch_types=[
        pltpu.VMEM_SHARED((N_NODES, DIM), jnp.float32),
        pltpu.VMEM((CH,), jnp.int32),
        pltpu.VMEM((CH,), jnp.int32),
        pltpu.VMEM((CH, DIM), jnp.float32),
    ],
    compiler_params=_sc_params,
)
def _sc_counts(didx_hbm, et_hbm, zeros_hbm, out_hbm, acc_sh, di_v, et_v,
               rows_v):
    c = lax.axis_index("c")
    s = lax.axis_index("s")
    base = (c * NS + s) * PER_W
    lanes = lax.iota(jnp.int32, 16)
    ones16 = jnp.ones((16,), jnp.float32)
    zeros16 = jnp.zeros((16,), jnp.float32)

    pltpu.sync_copy(zeros_hbm.at[pl.ds(0, CH)], rows_v)

    @pl.when(s == 0)
    def _():
        pltpu.sync_copy(zeros_hbm, acc_sh)

    plsc.subcore_barrier()

    @pl.loop(0, NCHUNK)
    def _(k):
        off = base + k * CH
        pltpu.sync_copy(didx_hbm.at[pl.ds(off, CH)], di_v)
        pltpu.sync_copy(et_hbm.at[pl.ds(off, CH)], et_v)

        @pl.loop(0, CH)
        def _(i):
            ri = jnp.full((16,), i, jnp.int32)
            t16 = plsc.load_gather(et_v, [ri])
            plsc.store_scatter(rows_v, [ri, t16 * 16 + lanes], ones16)

        pltpu.sync_copy(rows_v, acc_sh.at[di_v], add=True)

        @pl.loop(0, CH)
        def _(i):
            ri = jnp.full((16,), i, jnp.int32)
            t16 = plsc.load_gather(et_v, [ri])
            plsc.store_scatter(rows_v, [ri, t16 * 16 + lanes], zeros16)

    plsc.subcore_barrier()
    r0 = s * ROWS_PER_S
    pltpu.sync_copy(acc_sh.at[pl.ds(r0, ROWS_PER_S)],
                    out_hbm.at[c, pl.ds(r0, ROWS_PER_S)])

    @pl.when(s == 0)
    def _():
        rr = NS * ROWS_PER_S
        pltpu.sync_copy(acc_sh.at[pl.ds(rr, ROWS_REM)],
                        out_hbm.at[c, pl.ds(rr, ROWS_REM)])


@functools.partial(
    pl.kernel,
    out_type=jax.ShapeDtypeStruct((NC, N_NODES, DIM), jnp.float32),
    mesh=_mesh,
    scratch_types=[
        pltpu.VMEM_SHARED((N_NODES, DIM), jnp.float32),
        pltpu.VMEM((CH,), jnp.int32),
        pltpu.VMEM((CH,), jnp.int32),
        pltpu.VMEM((CH,), jnp.int32),
        pltpu.VMEM((CH,), jnp.int32),
        pltpu.VMEM((CH,), jnp.int32),
        pltpu.VMEM((CH,), jnp.int32),
        pltpu.VMEM((CH,), jnp.float32),
        pltpu.VMEM((CH,), jnp.float32),
        pltpu.VMEM((CH, DIM), jnp.float32),
        pltpu.VMEM((CH, DIM), jnp.float32),
        pltpu.SemaphoreType.DMA,
        pltpu.SemaphoreType.DMA,
        pltpu.SemaphoreType.DMA,
        pltpu.SemaphoreType.DMA,
    ],
    compiler_params=_sc_params,
)
def _sc_agg(hs_hbm, gidx_hbm, cidx_hbm, didx_hbm, stbl_hbm, zeros_hbm,
            out_hbm, acc_sh, gi0_v, ci0_v, di0_v, gi1_v, ci1_v, di1_v,
            sv0_v, sv1_v, rows0_v, rows1_v, semr0, semr1, sems0, sems1):
    c = lax.axis_index("c")
    s = lax.axis_index("s")
    base = (c * NS + s) * PER_W
    lanes = lax.iota(jnp.int32, 16)

    @pl.when(s == 0)
    def _():
        pltpu.sync_copy(zeros_hbm, acc_sh)

    plsc.subcore_barrier()

    def fetch(k, gi_v, ci_v, di_v, sv_v, rows_v, semr, sems):
        off = base + k * CH
        pltpu.sync_copy(gidx_hbm.at[pl.ds(off, CH)], gi_v)
        pltpu.async_copy(hs_hbm.at[gi_v], rows_v, semr)
        pltpu.sync_copy(cidx_hbm.at[pl.ds(off, CH)], ci_v)
        pltpu.async_copy(stbl_hbm.at[ci_v], sv_v, sems)
        pltpu.sync_copy(didx_hbm.at[pl.ds(off, CH)], di_v)

    def process(gi_v, ci_v, di_v, sv_v, rows_v, semr, sems):
        pltpu.make_async_copy(hs_hbm.at[gi_v], rows_v, semr).wait()
        pltpu.make_async_copy(stbl_hbm.at[ci_v], sv_v, sems).wait()
        @pl.loop(0, CH)
        def _(e):
            re = jnp.full((16,), e, jnp.int32)
            sc16 = plsc.load_gather(sv_v, [re])
            for j in range(DIM // 16):
                cj = j * 16 + lanes
                v = plsc.load_gather(rows_v, [re, cj])
                plsc.store_scatter(rows_v, [re, cj], v * sc16)

        pltpu.sync_copy(rows_v, acc_sh.at[di_v], add=True)

    fetch(0, gi0_v, ci0_v, di0_v, sv0_v, rows0_v, semr0, sems0)

    @pl.loop(0, (NCHUNK - 1) // 2)
    def _(j):
        fetch(2 * j + 1, gi1_v, ci1_v, di1_v, sv1_v, rows1_v, semr1, sems1)
        process(gi0_v, ci0_v, di0_v, sv0_v, rows0_v, semr0, sems0)
        fetch(2 * j + 2, gi0_v, ci0_v, di0_v, sv0_v, rows0_v, semr0, sems0)
        process(gi1_v, ci1_v, di1_v, sv1_v, rows1_v, semr1, sems1)

    process(gi0_v, ci0_v, di0_v, sv0_v, rows0_v, semr0, sems0)

    plsc.subcore_barrier()
    r0 = s * ROWS_PER_S
    pltpu.sync_copy(acc_sh.at[pl.ds(r0, ROWS_PER_S)],
                    out_hbm.at[c, pl.ds(r0, ROWS_PER_S)])

    @pl.when(s == 0)
    def _():
        rr = NS * ROWS_PER_S
        pltpu.sync_copy(acc_sh.at[pl.ds(rr, ROWS_REM)],
                        out_hbm.at[c, pl.ds(rr, ROWS_REM)])


def _mm_rel_body(x_ref, w_ref, o_ref):
    o_ref[...] = jnp.dot(x_ref[...], w_ref[0], preferred_element_type=jnp.float32)


_mm_rel = pl.pallas_call(
    _mm_rel_body,
    grid=(NREL,),
    in_specs=[pl.BlockSpec((N_NODES, DIM), lambda r: (0, 0)),
              pl.BlockSpec((1, DIM, DIM), lambda r: (r, 0, 0))],
    out_specs=pl.BlockSpec((N_NODES, DIM), lambda r: (r, 0)),
    out_shape=jax.ShapeDtypeStruct((NREL * N_NODES, DIM), jnp.float32),
)


def _recip_body(c_ref, o_ref):
    o_ref[...] = 1.0 / jnp.maximum(c_ref[0] + c_ref[1], 1.0)


_recip = pl.pallas_call(
    _recip_body,
    in_specs=[pl.BlockSpec((NC, N_NODES, DIM), lambda: (0, 0, 0))],
    out_specs=pl.BlockSpec((N_NODES, DIM), lambda: (0, 0)),
    out_shape=jax.ShapeDtypeStruct((N_NODES, DIM), jnp.float32),
)


def _mid_body(x_ref, wr_ref, b_ref, a0_ref, a1_ref, wrel_ref, h_ref, hs_ref):
    r = pl.program_id(0)

    @pl.when(r == 0)
    def _():
        h_ref[...] = jnp.maximum(
            jnp.dot(x_ref[...], wr_ref[...], preferred_element_type=jnp.float32)
            + b_ref[...] + a0_ref[...] + a1_ref[...], 0.0)

    hs_ref[...] = jnp.dot(h_ref[...], wrel_ref[0],
                          preferred_element_type=jnp.float32)


_mid = pl.pallas_call(
    _mid_body,
    grid=(NREL,),
    in_specs=[
        pl.BlockSpec((N_NODES, DIM), lambda r: (0, 0)),
        pl.BlockSpec((DIM, DIM), lambda r: (0, 0)),
        pl.BlockSpec((1, DIM), lambda r: (0, 0)),
        pl.BlockSpec((N_NODES, DIM), lambda r: (0, 0)),
        pl.BlockSpec((N_NODES, DIM), lambda r: (0, 0)),
        pl.BlockSpec((1, DIM, DIM), lambda r: (r, 0, 0)),
    ],
    out_specs=[pl.BlockSpec((N_NODES, DIM), lambda r: (0, 0)),
               pl.BlockSpec((N_NODES, DIM), lambda r: (r, 0))],
    out_shape=[jax.ShapeDtypeStruct((N_NODES, DIM), jnp.float32),
               jax.ShapeDtypeStruct((NREL * N_NODES, DIM), jnp.float32)],
)


def _final_body(h1_ref, wr_ref, b_ref, a0_ref, a1_ref, g_ref, lw_ref, lb_ref,
                o_ref):
    h2 = jnp.maximum(
        jnp.dot(h1_ref[...], wr_ref[...], preferred_element_type=jnp.float32)
        + b_ref[...] + a0_ref[...] + a1_ref[...], 0.0)
    gids = lax.broadcasted_iota(jnp.int32, (1, NGRAPH), 1)
    p = (g_ref[...] == gids).astype(jnp.float32)
    sums = lax.dot_general(p, h2, (((0,), (0,)), ((), ())),
                           preferred_element_type=jnp.float32)
    cnts = jnp.sum(p, axis=0)
    pooled = sums / jnp.maximum(cnts, 1.0)[:, None]
    o_ref[...] = (jnp.dot(pooled, lw_ref[...], preferred_element_type=jnp.float32)
                  + lb_ref[...])


_final = pl.pallas_call(
    _final_body,
    in_specs=[
        pl.BlockSpec((N_NODES, DIM), lambda: (0, 0)),
        pl.BlockSpec((DIM, DIM), lambda: (0, 0)),
        pl.BlockSpec((1, DIM), lambda: (0, 0)),
        pl.BlockSpec((N_NODES, DIM), lambda: (0, 0)),
        pl.BlockSpec((N_NODES, DIM), lambda: (0, 0)),
        pl.BlockSpec((N_NODES, 1), lambda: (0, 0)),
        pl.BlockSpec((DIM, CLASSES), lambda: (0, 0)),
        pl.BlockSpec((1, CLASSES), lambda: (0, 0)),
    ],
    out_specs=pl.BlockSpec((NGRAPH, CLASSES), lambda: (0, 0)),
    out_shape=jax.ShapeDtypeStruct((NGRAPH, CLASSES), jnp.float32),
)


def kernel(x, edge_index, edge_type, batch, w1_rel, w1_root, b1,
           w2_rel, w2_root, b2, lin_w, lin_b):
    src = edge_index[0].astype(jnp.int32)
    dst = edge_index[1].astype(jnp.int32)
    et = edge_type.astype(jnp.int32)
    gidx = et * N_NODES + src
    cidx = et * N_NODES + dst
    zeros_agg = jnp.zeros((N_NODES, DIM), jnp.float32)

    cnt_parts = _sc_counts(dst, et, zeros_agg)
    s2d = _recip(cnt_parts)
    stbl = jnp.concatenate([s2d[:, 16 * r] for r in range(NREL)])
    hs1 = _mm_rel(x, w1_rel)
    agg1 = _sc_agg(hs1, gidx, cidx, dst, stbl, zeros_agg)
    h1, hs2 = _mid(x, w1_root, b1.reshape(1, DIM), agg1[0], agg1[1], w2_rel)
    agg2 = _sc_agg(hs2, gidx, cidx, dst, stbl, zeros_agg)
    return _final(h1, w2_root, b2.reshape(1, DIM), agg2[0], agg2[1],
                  batch.astype(jnp.int32).reshape(N_NODES, 1),
                  lin_w, lin_b.reshape(1, CLASSES))


# speedup vs baseline: 6.5131x; 1.0993x over previous
import dataclasses
import functools

import jax
import jax.numpy as jnp
from jax import lax
from jax.experimental import pallas as pl
from jax.experimental.pallas import tpu as pltpu
from jax.experimental.pallas import tpu_sc as plsc

N_NODES = 10000
N_EDGES = 320000
DIM = 128
NREL = 3
NGRAPH = 128
CLASSES = 64

NC = 2
NS = 16
PER_W = N_EDGES // (NC * NS)
CH = 80
NCHUNK = PER_W // CH
ROWS_PER_S = (N_NODES // NS) // 8 * 8
ROWS_REM = N_NODES - NS * ROWS_PER_S

_mesh = plsc.VectorSubcoreMesh(core_axis_name="c", subcore_axis_name="s")

_sc_params = pltpu.CompilerParams()
if "needs_layout_passes" in pltpu.CompilerParams.__dataclass_fields__:
    _sc_params = dataclasses.replace(_sc_params, needs_layout_passes=False)


@functools.partial(
    pl.kernel,
    out_type=jax.ShapeDtypeStruct((NC, N_NODES, DIM), jnp.float32),
    mesh=_mesh,
    scratch_types=[
        pltpu.VMEM_SHARED((N_NODES, DIM), jnp.float32),
        pltpu.VMEM((CH,), jnp.int32),
        pltpu.VMEM((CH,), jnp.int32),
        pltpu.VMEM((CH, DIM), jnp.float32),
    ],
    compiler_params=_sc_params,
)
def _sc_counts(didx_hbm, et_hbm, zeros_hbm, out_hbm, acc_sh, di_v, et_v,
               rows_v):
    c = lax.axis_index("c")
    s = lax.axis_index("s")
    base = (c * NS + s) * PER_W
    lanes = lax.iota(jnp.int32, 16)
    ones16 = jnp.ones((16,), jnp.float32)
    zeros16 = jnp.zeros((16,), jnp.float32)

    pltpu.sync_copy(zeros_hbm.at[pl.ds(0, CH)], rows_v)

    @pl.when(s == 0)
    def _():
        pltpu.sync_copy(zeros_hbm, acc_sh)

    plsc.subcore_barrier()

    @pl.loop(0, NCHUNK)
    def _(k):
        off = base + k * CH
        pltpu.sync_copy(didx_hbm.at[pl.ds(off, CH)], di_v)
        pltpu.sync_copy(et_hbm.at[pl.ds(off, CH)], et_v)

        for g in range(CH // 16):
            e16 = g * 16 + lanes
            t16 = et_v[pl.ds(g * 16, 16)]
            plsc.store_scatter(rows_v, [e16, t16 * 16], ones16)

        pltpu.sync_copy(rows_v, acc_sh.at[di_v], add=True)

        for g in range(CH // 16):
            e16 = g * 16 + lanes
            t16 = et_v[pl.ds(g * 16, 16)]
            plsc.store_scatter(rows_v, [e16, t16 * 16], zeros16)

    plsc.subcore_barrier()
    r0 = s * ROWS_PER_S
    pltpu.sync_copy(acc_sh.at[pl.ds(r0, ROWS_PER_S)],
                    out_hbm.at[c, pl.ds(r0, ROWS_PER_S)])

    @pl.when(s == 0)
    def _():
        rr = NS * ROWS_PER_S
        pltpu.sync_copy(acc_sh.at[pl.ds(rr, ROWS_REM)],
                        out_hbm.at[c, pl.ds(rr, ROWS_REM)])


@functools.partial(
    pl.kernel,
    out_type=jax.ShapeDtypeStruct((NC, N_NODES, DIM), jnp.float32),
    mesh=_mesh,
    scratch_types=[
        pltpu.VMEM_SHARED((N_NODES, DIM), jnp.float32),
        pltpu.VMEM((CH,), jnp.int32),
        pltpu.VMEM((CH,), jnp.int32),
        pltpu.VMEM((CH,), jnp.int32),
        pltpu.VMEM((CH,), jnp.int32),
        pltpu.VMEM((CH,), jnp.int32),
        pltpu.VMEM((CH,), jnp.int32),
        pltpu.VMEM((CH,), jnp.float32),
        pltpu.VMEM((CH,), jnp.float32),
        pltpu.VMEM((CH, DIM), jnp.float32),
        pltpu.VMEM((CH, DIM), jnp.float32),
        pltpu.SemaphoreType.DMA,
        pltpu.SemaphoreType.DMA,
        pltpu.SemaphoreType.DMA,
        pltpu.SemaphoreType.DMA,
    ],
    compiler_params=_sc_params,
)
def _sc_agg(hs_hbm, gidx_hbm, cidx_hbm, didx_hbm, stbl_hbm, zeros_hbm,
            out_hbm, acc_sh, gi0_v, ci0_v, di0_v, gi1_v, ci1_v, di1_v,
            sv0_v, sv1_v, rows0_v, rows1_v, semr0, semr1, sems0, sems1):
    c = lax.axis_index("c")
    s = lax.axis_index("s")
    base = (c * NS + s) * PER_W
    lanes = lax.iota(jnp.int32, 16)

    @pl.when(s == 0)
    def _():
        pltpu.sync_copy(zeros_hbm, acc_sh)

    plsc.subcore_barrier()

    def fetch(k, gi_v, ci_v, di_v, sv_v, rows_v, semr, sems):
        off = base + k * CH
        pltpu.sync_copy(gidx_hbm.at[pl.ds(off, CH)], gi_v)
        pltpu.async_copy(hs_hbm.at[gi_v], rows_v, semr)
        pltpu.sync_copy(cidx_hbm.at[pl.ds(off, CH)], ci_v)
        pltpu.async_copy(stbl_hbm.at[ci_v], sv_v, sems)
        pltpu.sync_copy(didx_hbm.at[pl.ds(off, CH)], di_v)

    def process(gi_v, ci_v, di_v, sv_v, rows_v, semr, sems):
        pltpu.make_async_copy(hs_hbm.at[gi_v], rows_v, semr).wait()
        pltpu.make_async_copy(stbl_hbm.at[ci_v], sv_v, sems).wait()
        @pl.loop(0, CH)
        def _(e):
            re = jnp.full((16,), e, jnp.int32)
            sc16 = plsc.load_gather(sv_v, [re])
            for j in range(DIM // 16):
                cj = j * 16 + lanes
                v = plsc.load_gather(rows_v, [re, cj])
                plsc.store_scatter(rows_v, [re, cj], v * sc16)

        pltpu.sync_copy(rows_v, acc_sh.at[di_v], add=True)

    fetch(0, gi0_v, ci0_v, di0_v, sv0_v, rows0_v, semr0, sems0)

    @pl.loop(0, (NCHUNK - 1) // 2)
    def _(j):
        fetch(2 * j + 1, gi1_v, ci1_v, di1_v, sv1_v, rows1_v, semr1, sems1)
        process(gi0_v, ci0_v, di0_v, sv0_v, rows0_v, semr0, sems0)
        fetch(2 * j + 2, gi0_v, ci0_v, di0_v, sv0_v, rows0_v, semr0, sems0)
        process(gi1_v, ci1_v, di1_v, sv1_v, rows1_v, semr1, sems1)

    process(gi0_v, ci0_v, di0_v, sv0_v, rows0_v, semr0, sems0)

    plsc.subcore_barrier()
    r0 = s * ROWS_PER_S
    pltpu.sync_copy(acc_sh.at[pl.ds(r0, ROWS_PER_S)],
                    out_hbm.at[c, pl.ds(r0, ROWS_PER_S)])

    @pl.when(s == 0)
    def _():
        rr = NS * ROWS_PER_S
        pltpu.sync_copy(acc_sh.at[pl.ds(rr, ROWS_REM)],
                        out_hbm.at[c, pl.ds(rr, ROWS_REM)])


def _mm_rel_body(x_ref, w_ref, o_ref):
    o_ref[...] = jnp.dot(x_ref[...], w_ref[0], preferred_element_type=jnp.float32)


_mm_rel = pl.pallas_call(
    _mm_rel_body,
    grid=(NREL,),
    in_specs=[pl.BlockSpec((N_NODES, DIM), lambda r: (0, 0)),
              pl.BlockSpec((1, DIM, DIM), lambda r: (r, 0, 0))],
    out_specs=pl.BlockSpec((N_NODES, DIM), lambda r: (r, 0)),
    out_shape=jax.ShapeDtypeStruct((NREL * N_NODES, DIM), jnp.float32),
)


def _recip_body(c_ref, o_ref):
    o_ref[...] = 1.0 / jnp.maximum(c_ref[0] + c_ref[1], 1.0)


_recip = pl.pallas_call(
    _recip_body,
    in_specs=[pl.BlockSpec((NC, N_NODES, DIM), lambda: (0, 0, 0))],
    out_specs=pl.BlockSpec((N_NODES, DIM), lambda: (0, 0)),
    out_shape=jax.ShapeDtypeStruct((N_NODES, DIM), jnp.float32),
)


def _mid_body(x_ref, wr_ref, b_ref, a0_ref, a1_ref, wrel_ref, h_ref, hs_ref):
    r = pl.program_id(0)

    @pl.when(r == 0)
    def _():
        h_ref[...] = jnp.maximum(
            jnp.dot(x_ref[...], wr_ref[...], preferred_element_type=jnp.float32)
            + b_ref[...] + a0_ref[...] + a1_ref[...], 0.0)

    hs_ref[...] = jnp.dot(h_ref[...], wrel_ref[0],
                          preferred_element_type=jnp.float32)


_mid = pl.pallas_call(
    _mid_body,
    grid=(NREL,),
    in_specs=[
        pl.BlockSpec((N_NODES, DIM), lambda r: (0, 0)),
        pl.BlockSpec((DIM, DIM), lambda r: (0, 0)),
        pl.BlockSpec((1, DIM), lambda r: (0, 0)),
        pl.BlockSpec((N_NODES, DIM), lambda r: (0, 0)),
        pl.BlockSpec((N_NODES, DIM), lambda r: (0, 0)),
        pl.BlockSpec((1, DIM, DIM), lambda r: (r, 0, 0)),
    ],
    out_specs=[pl.BlockSpec((N_NODES, DIM), lambda r: (0, 0)),
               pl.BlockSpec((N_NODES, DIM), lambda r: (r, 0))],
    out_shape=[jax.ShapeDtypeStruct((N_NODES, DIM), jnp.float32),
               jax.ShapeDtypeStruct((NREL * N_NODES, DIM), jnp.float32)],
)


def _final_body(h1_ref, wr_ref, b_ref, a0_ref, a1_ref, g_ref, lw_ref, lb_ref,
                o_ref):
    h2 = jnp.maximum(
        jnp.dot(h1_ref[...], wr_ref[...], preferred_element_type=jnp.float32)
        + b_ref[...] + a0_ref[...] + a1_ref[...], 0.0)
    gids = lax.broadcasted_iota(jnp.int32, (1, NGRAPH), 1)
    p = (g_ref[...] == gids).astype(jnp.float32)
    sums = lax.dot_general(p, h2, (((0,), (0,)), ((), ())),
                           preferred_element_type=jnp.float32)
    cnts = jnp.sum(p, axis=0)
    pooled = sums / jnp.maximum(cnts, 1.0)[:, None]
    o_ref[...] = (jnp.dot(pooled, lw_ref[...], preferred_element_type=jnp.float32)
                  + lb_ref[...])


_final = pl.pallas_call(
    _final_body,
    in_specs=[
        pl.BlockSpec((N_NODES, DIM), lambda: (0, 0)),
        pl.BlockSpec((DIM, DIM), lambda: (0, 0)),
        pl.BlockSpec((1, DIM), lambda: (0, 0)),
        pl.BlockSpec((N_NODES, DIM), lambda: (0, 0)),
        pl.BlockSpec((N_NODES, DIM), lambda: (0, 0)),
        pl.BlockSpec((N_NODES, 1), lambda: (0, 0)),
        pl.BlockSpec((DIM, CLASSES), lambda: (0, 0)),
        pl.BlockSpec((1, CLASSES), lambda: (0, 0)),
    ],
    out_specs=pl.BlockSpec((NGRAPH, CLASSES), lambda: (0, 0)),
    out_shape=jax.ShapeDtypeStruct((NGRAPH, CLASSES), jnp.float32),
)


def kernel(x, edge_index, edge_type, batch, w1_rel, w1_root, b1,
           w2_rel, w2_root, b2, lin_w, lin_b):
    src = edge_index[0].astype(jnp.int32)
    dst = edge_index[1].astype(jnp.int32)
    et = edge_type.astype(jnp.int32)
    gidx = et * N_NODES + src
    cidx = et * N_NODES + dst
    zeros_agg = jnp.zeros((N_NODES, DIM), jnp.float32)

    cnt_parts = _sc_counts(dst, et, zeros_agg)
    s2d = _recip(cnt_parts)
    stbl = jnp.concatenate([s2d[:, 16 * r] for r in range(NREL)])
    hs1 = _mm_rel(x, w1_rel)
    agg1 = _sc_agg(hs1, gidx, cidx, dst, stbl, zeros_agg)
    h1, hs2 = _mid(x, w1_root, b1.reshape(1, DIM), agg1[0], agg1[1], w2_rel)
    agg2 = _sc_agg(hs2, gidx, cidx, dst, stbl, zeros_agg)
    return _final(h1, w2_root, b2.reshape(1, DIM), agg2[0], agg2[1],
                  batch.astype(jnp.int32).reshape(N_NODES, 1),
                  lin_w, lin_b.reshape(1, CLASSES))
